# Initial kernel scaffold; baseline (speedup 1.0000x reference)
import jax
import jax.numpy as jnp
from jax.experimental import pallas as pl


def kernel(x, edge_index, edge_weight, W, b):
    raise NotImplementedError("write your pallas kernel here")


# speedup vs baseline: 15.7885x; 15.7885x over previous
import functools

import jax
import jax.numpy as jnp
from jax import lax
from jax.experimental import pallas as pl
from jax.experimental.pallas import tpu as pltpu
from jax.experimental.pallas import tpu_sc as plsc

NC = 2
NS = 16
L = 16
CHUNK = 128


def _deg_body(cpt, nch, spt, cols_hbm, ew_hbm, deg_out, col_v, ew_v,
              stripe_v, deg_sh):
    c = lax.axis_index("c")
    s = lax.axis_index("s")
    wid = s * NC + c
    zero = jnp.zeros((L,), jnp.float32)

    def zfill(i, carry):
        stripe_v[pl.ds(i * L, L)] = zero
        return carry

    lax.fori_loop(0, spt // L, zfill, 0)
    pltpu.sync_copy(stripe_v, deg_sh.at[pl.ds(s * spt, spt)])
    plsc.subcore_barrier()

    def chunk(i, carry):
        base = wid * cpt + i * CHUNK
        pltpu.sync_copy(cols_hbm.at[pl.ds(base, CHUNK)], col_v)
        pltpu.sync_copy(ew_hbm.at[pl.ds(base, CHUNK)], ew_v)
        pltpu.sync_copy(ew_v, deg_sh.at[col_v], add=True)
        return carry

    lax.fori_loop(0, nch, chunk, 0)
    plsc.subcore_barrier()
    n_pad = deg_sh.shape[0]
    pltpu.sync_copy(deg_sh.at[pl.ds(s * spt, spt)],
                    deg_out.at[pl.ds(c * n_pad + s * spt, spt)])


def _acc_body(cpt, nch, spt, d_out, g_hbm, rows_hbm, cols_hbm, ew_hbm,
              acc_out, row_v, col_v, ew_v, rows_v, acc_sh, gsem):
    c = lax.axis_index("c")
    s = lax.axis_index("s")
    wid = s * NC + c
    zero = jnp.zeros((L,), jnp.float32)
    nvec = d_out // L

    def zfill(i, carry):
        for r in range(nvec):
            rows_v[i, pl.ds(r * L, L)] = zero
        return carry

    lax.fori_loop(0, CHUNK, zfill, 0)

    def zcopy(k, carry):
        pltpu.sync_copy(rows_v, acc_sh.at[pl.ds(s * spt + k * CHUNK, CHUNK)])
        return carry

    lax.fori_loop(0, spt // CHUNK, zcopy, 0)
    plsc.subcore_barrier()

    def chunk(i, carry):
        base = wid * cpt + i * CHUNK
        pltpu.sync_copy(rows_hbm.at[pl.ds(base, CHUNK)], row_v)
        pltpu.sync_copy(ew_hbm.at[pl.ds(base, CHUNK)], ew_v)
        pltpu.sync_copy(cols_hbm.at[pl.ds(base, CHUNK)], col_v)
        pltpu.async_copy(g_hbm.at[row_v], rows_v, gsem).wait()

        def scale(gi, carry2):
            wv = ew_v[pl.ds(gi * L, L)]
            for j in range(L):
                w_e = wv[j]
                eidx = gi * L + j
                for r in range(nvec):
                    sl = pl.ds(r * L, L)
                    rows_v[eidx, sl] = rows_v[eidx, sl] * w_e
            return carry2

        lax.fori_loop(0, CHUNK // L, scale, 0)
        pltpu.sync_copy(rows_v, acc_sh.at[col_v], add=True)
        return carry

    lax.fori_loop(0, nch, chunk, 0)
    plsc.subcore_barrier()
    n_pad = acc_sh.shape[0]
    pltpu.sync_copy(acc_sh.at[pl.ds(s * spt, spt)],
                    acc_out.at[pl.ds(c * n_pad + s * spt, spt)])


def _tc1_body(x_ref, w_ref, deg_ref, g_ref, dinv_ref):
    d = deg_ref[0] + deg_ref[1] + 1.0
    di = lax.rsqrt(d)
    h = jnp.dot(x_ref[...], w_ref[...], preferred_element_type=jnp.float32)
    g_ref[...] = h * di
    dinv_ref[...] = di


def _tc2_body(a_ref, g_ref, dinv_ref, b_ref, o_ref):
    o_ref[...] = (a_ref[0] + a_ref[1] + g_ref[...]) * dinv_ref[...] + b_ref[...]


def kernel(x, edge_index, edge_weight, W, b):
    f32 = jnp.float32
    n, d_in = x.shape
    d_out = W.shape[1]
    e = edge_index.shape[1]

    nw = NC * NS
    n_pad = -(-n // 512) * 512
    spt = n_pad // NS
    cpt = -(-e // (nw * CHUNK)) * CHUNK
    nch = cpt // CHUNK
    e_pad = nw * cpt
    pad = e_pad - e

    row = edge_index[0]
    col = edge_index[1]
    pad_idx = jnp.arange(pad, dtype=jnp.int32) % n
    rows_p = jnp.concatenate([row, pad_idx])
    cols_p = jnp.concatenate([col, pad_idx])
    ew_p = jnp.concatenate([edge_weight, jnp.zeros((pad,), f32)])
    x_pad = jnp.pad(x, ((0, n_pad - n), (0, 0)))

    mesh = plsc.VectorSubcoreMesh(core_axis_name="c", subcore_axis_name="s")

    deg_fn = pl.kernel(
        functools.partial(_deg_body, cpt, nch, spt),
        out_type=jax.ShapeDtypeStruct((NC * n_pad,), f32),
        mesh=mesh,
        scratch_types=[
            pltpu.VMEM((CHUNK,), jnp.int32),
            pltpu.VMEM((CHUNK,), f32),
            pltpu.VMEM((spt,), f32),
            pltpu.VMEM_SHARED((n_pad,), f32),
        ],
    )
    deg_flat = deg_fn(cols_p, ew_p)
    deg3 = deg_flat.reshape(NC, n_pad, 1)

    rb = 1024
    grid = (n_pad // rb,)
    g, dinv = pl.pallas_call(
        _tc1_body,
        grid=grid,
        in_specs=[
            pl.BlockSpec((rb, d_in), lambda i: (i, 0)),
            pl.BlockSpec((d_in, d_out), lambda i: (0, 0)),
            pl.BlockSpec((NC, rb, 1), lambda i: (0, i, 0)),
        ],
        out_specs=[
            pl.BlockSpec((rb, d_out), lambda i: (i, 0)),
            pl.BlockSpec((rb, 1), lambda i: (i, 0)),
        ],
        out_shape=[
            jax.ShapeDtypeStruct((n_pad, d_out), f32),
            jax.ShapeDtypeStruct((n_pad, 1), f32),
        ],
    )(x_pad, W, deg3)

    acc_fn = pl.kernel(
        functools.partial(_acc_body, cpt, nch, spt, d_out),
        out_type=jax.ShapeDtypeStruct((NC * n_pad, d_out), f32),
        mesh=mesh,
        scratch_types=[
            pltpu.VMEM((CHUNK,), jnp.int32),
            pltpu.VMEM((CHUNK,), jnp.int32),
            pltpu.VMEM((CHUNK,), f32),
            pltpu.VMEM((CHUNK, d_out), f32),
            pltpu.VMEM_SHARED((n_pad, d_out), f32),
            pltpu.SemaphoreType.DMA,
        ],
    )
    acc = acc_fn(g, rows_p, cols_p, ew_p).reshape(NC, n_pad, d_out)

    out_pad = pl.pallas_call(
        _tc2_body,
        grid=grid,
        in_specs=[
            pl.BlockSpec((NC, rb, d_out), lambda i: (0, i, 0)),
            pl.BlockSpec((rb, d_out), lambda i: (i, 0)),
            pl.BlockSpec((rb, 1), lambda i: (i, 0)),
            pl.BlockSpec((1, d_out), lambda i: (0, 0)),
        ],
        out_specs=pl.BlockSpec((rb, d_out), lambda i: (i, 0)),
        out_shape=jax.ShapeDtypeStruct((n_pad, d_out), f32),
    )(acc, g, dinv, b.reshape(1, d_out))

    return out_pad[:n]


# speedup vs baseline: 30.0518x; 1.9034x over previous
import functools

import jax
import jax.numpy as jnp
from jax import lax
from jax.experimental import pallas as pl
from jax.experimental.pallas import tpu as pltpu
from jax.experimental.pallas import tpu_sc as plsc

NC = 2
NS = 16
L = 16
CHUNK = 128
BLK = 8
KB = CHUNK * BLK


def _deg_body(cbt, nblk, spt, cols_hbm, ew_hbm, deg_out, col_v, ew_v,
              stripe_v, deg_sh):
    c = lax.axis_index("c")
    s = lax.axis_index("s")
    wid = s * NC + c
    zero = jnp.zeros((L,), jnp.float32)

    def zfill(i, carry):
        stripe_v[pl.ds(i * L, L)] = zero
        return carry

    lax.fori_loop(0, spt // L, zfill, 0)
    pltpu.sync_copy(stripe_v, deg_sh.at[pl.ds(s * spt, spt)])
    plsc.subcore_barrier()

    def block(i, carry):
        base = wid * cbt + i * BLK
        pltpu.sync_copy(cols_hbm.at[pl.ds(base, BLK)], col_v)
        pltpu.sync_copy(ew_hbm.at[pl.ds(base, BLK)], ew_v)
        for j in range(BLK):
            pltpu.sync_copy(ew_v.at[j], deg_sh.at[col_v.at[j]], add=True)
        return carry

    lax.fori_loop(0, nblk, block, 0)
    plsc.subcore_barrier()
    n_pad = deg_sh.shape[0]
    pltpu.sync_copy(deg_sh.at[pl.ds(s * spt, spt)],
                    deg_out.at[pl.ds(c * n_pad + s * spt, spt)])


def _acc_body(cbt, nblk, spt, d_out, g_hbm, rows_hbm, cols_hbm, ew_hbm,
              acc_out, row_v, col_v, ew_v, rows_v, acc_sh, gsem):
    c = lax.axis_index("c")
    s = lax.axis_index("s")
    wid = s * NC + c
    zero = jnp.zeros((L,), jnp.float32)
    nvec = d_out // L

    def zfill(i, carry):
        for r in range(nvec):
            rows_v[0, i, pl.ds(r * L, L)] = zero
        return carry

    lax.fori_loop(0, CHUNK, zfill, 0)

    def zcopy(k, carry):
        pltpu.sync_copy(rows_v.at[0], acc_sh.at[pl.ds(s * spt + k * CHUNK, CHUNK)])
        return carry

    lax.fori_loop(0, spt // CHUNK, zcopy, 0)
    plsc.subcore_barrier()

    def scale(buf, j):
        def grp(gi, carry):
            wv = ew_v[j, pl.ds(gi * L, L)]
            for u in range(L):
                w_e = wv[u]
                eidx = gi * L + u
                for r in range(nvec):
                    sl = pl.ds(r * L, L)
                    rows_v[buf, eidx, sl] = rows_v[buf, eidx, sl] * w_e
            return carry

        lax.fori_loop(0, CHUNK // L, grp, 0)

    def block(i, carry):
        base = wid * cbt + i * BLK
        pltpu.sync_copy(rows_hbm.at[pl.ds(base, BLK)], row_v)
        pltpu.sync_copy(ew_hbm.at[pl.ds(base, BLK)], ew_v)
        pltpu.sync_copy(cols_hbm.at[pl.ds(base, BLK)], col_v)
        d = pltpu.async_copy(g_hbm.at[row_v.at[0]], rows_v.at[0], gsem)
        for j in range(BLK):
            d.wait()
            if j + 1 < BLK:
                d = pltpu.async_copy(g_hbm.at[row_v.at[j + 1]],
                                     rows_v.at[(j + 1) % 2], gsem)
            scale(j % 2, j)
            pltpu.sync_copy(rows_v.at[j % 2], acc_sh.at[col_v.at[j]], add=True)
        return carry

    lax.fori_loop(0, nblk, block, 0)
    plsc.subcore_barrier()
    n_pad = acc_sh.shape[0]
    pltpu.sync_copy(acc_sh.at[pl.ds(s * spt, spt)],
                    acc_out.at[pl.ds(c * n_pad + s * spt, spt)])


def _tc1_body(x_ref, w_ref, deg_ref, g_ref, dinv_ref):
    d = deg_ref[0] + deg_ref[1] + 1.0
    di = lax.rsqrt(d)
    h = jnp.dot(x_ref[...], w_ref[...], preferred_element_type=jnp.float32)
    g_ref[...] = h * di
    dinv_ref[...] = di


def _tc2_body(a_ref, g_ref, dinv_ref, b_ref, o_ref):
    o_ref[...] = (a_ref[0] + a_ref[1] + g_ref[...]) * dinv_ref[...] + b_ref[...]


def kernel(x, edge_index, edge_weight, W, b):
    f32 = jnp.float32
    n, d_in = x.shape
    d_out = W.shape[1]
    e = edge_index.shape[1]

    nw = NC * NS
    n_pad = -(-n // 512) * 512
    spt = n_pad // NS
    cpt = -(-e // (nw * KB)) * KB
    cbt = cpt // CHUNK
    nblk = cpt // KB
    e_pad = nw * cpt
    pad = e_pad - e

    row = edge_index[0]
    col = edge_index[1]
    pad_idx = jnp.arange(pad, dtype=jnp.int32) % n
    rows_p = jnp.concatenate([row, pad_idx]).reshape(-1, CHUNK)
    cols_p = jnp.concatenate([col, pad_idx]).reshape(-1, CHUNK)
    ew_p = jnp.concatenate([edge_weight, jnp.zeros((pad,), f32)]).reshape(-1, CHUNK)
    x_pad = jnp.pad(x, ((0, n_pad - n), (0, 0)))

    mesh = plsc.VectorSubcoreMesh(core_axis_name="c", subcore_axis_name="s")

    deg_fn = pl.kernel(
        functools.partial(_deg_body, cbt, nblk, spt),
        out_type=jax.ShapeDtypeStruct((NC * n_pad,), f32),
        mesh=mesh,
        scratch_types=[
            pltpu.VMEM((BLK, CHUNK), jnp.int32),
            pltpu.VMEM((BLK, CHUNK), f32),
            pltpu.VMEM((spt,), f32),
            pltpu.VMEM_SHARED((n_pad,), f32),
        ],
    )
    deg_flat = deg_fn(cols_p, ew_p)
    deg3 = deg_flat.reshape(NC, n_pad, 1)

    rb = 1024
    grid = (n_pad // rb,)
    g, dinv = pl.pallas_call(
        _tc1_body,
        grid=grid,
        in_specs=[
            pl.BlockSpec((rb, d_in), lambda i: (i, 0)),
            pl.BlockSpec((d_in, d_out), lambda i: (0, 0)),
            pl.BlockSpec((NC, rb, 1), lambda i: (0, i, 0)),
        ],
        out_specs=[
            pl.BlockSpec((rb, d_out), lambda i: (i, 0)),
            pl.BlockSpec((rb, 1), lambda i: (i, 0)),
        ],
        out_shape=[
            jax.ShapeDtypeStruct((n_pad, d_out), f32),
            jax.ShapeDtypeStruct((n_pad, 1), f32),
        ],
    )(x_pad, W, deg3)

    acc_fn = pl.kernel(
        functools.partial(_acc_body, cbt, nblk, spt, d_out),
        out_type=jax.ShapeDtypeStruct((NC * n_pad, d_out), f32),
        mesh=mesh,
        scratch_types=[
            pltpu.VMEM((BLK, CHUNK), jnp.int32),
            pltpu.VMEM((BLK, CHUNK), jnp.int32),
            pltpu.VMEM((BLK, CHUNK), f32),
            pltpu.VMEM((2, CHUNK, d_out), f32),
            pltpu.VMEM_SHARED((n_pad, d_out), f32),
            pltpu.SemaphoreType.DMA,
        ],
    )
    acc = acc_fn(g, rows_p, cols_p, ew_p).reshape(NC, n_pad, d_out)

    out_pad = pl.pallas_call(
        _tc2_body,
        grid=grid,
        in_specs=[
            pl.BlockSpec((NC, rb, d_out), lambda i: (0, i, 0)),
            pl.BlockSpec((rb, d_out), lambda i: (i, 0)),
            pl.BlockSpec((rb, 1), lambda i: (i, 0)),
            pl.BlockSpec((1, d_out), lambda i: (0, 0)),
        ],
        out_specs=pl.BlockSpec((rb, d_out), lambda i: (i, 0)),
        out_shape=jax.ShapeDtypeStruct((n_pad, d_out), f32),
    )(acc, g, dinv, b.reshape(1, d_out))

    return out_pad[:n]


# speedup vs baseline: 33.4723x; 1.1138x over previous
import functools

import jax
import jax.numpy as jnp
from jax import lax
from jax.experimental import pallas as pl
from jax.experimental.pallas import tpu as pltpu
from jax.experimental.pallas import tpu_sc as plsc

NC = 2
NS = 16
L = 16
CHUNK = 128
BLK = 8
KB = CHUNK * BLK


def _deg_body(cbt, spt, cols_hbm, ew_hbm, deg_out, col_a, ew_a,
              stripe_v, deg_sh, ssem):
    c = lax.axis_index("c")
    s = lax.axis_index("s")
    wid = s * NC + c
    zero = jnp.zeros((L,), jnp.float32)

    def zfill(i, carry):
        stripe_v[pl.ds(i * L, L)] = zero
        return carry

    lax.fori_loop(0, spt // L, zfill, 0)
    pltpu.sync_copy(stripe_v, deg_sh.at[pl.ds(s * spt, spt)])
    pltpu.sync_copy(cols_hbm.at[pl.ds(wid * cbt, cbt)], col_a)
    pltpu.sync_copy(ew_hbm.at[pl.ds(wid * cbt, cbt)], ew_a)
    plsc.subcore_barrier()

    lag = 8

    def drain():
        pltpu.make_async_copy(ew_hbm.at[0], ew_a.at[0], ssem).wait()

    def chunk(j, carry):
        pltpu.async_copy(ew_a.at[j], deg_sh.at[col_a.at[j]], ssem, add=True)
        pl.when(j >= lag)(drain)
        return carry

    lax.fori_loop(0, cbt, chunk, 0)

    def drain_i(i, carry):
        drain()
        return carry

    lax.fori_loop(0, lag, drain_i, 0)
    plsc.subcore_barrier()
    n_pad = deg_sh.shape[0]
    pltpu.sync_copy(deg_sh.at[pl.ds(s * spt, spt)],
                    deg_out.at[pl.ds(c * n_pad + s * spt, spt)])


def _acc_body(cbt, spt, d_out, g_hbm, rows_hbm, cols_hbm, ew_hbm,
              acc_out, row_b, col_b, ew_b, rows_v, acc_sh, gsem, ssem, isem):
    c = lax.axis_index("c")
    s = lax.axis_index("s")
    wid = s * NC + c
    nblk = cbt // BLK
    zero = jnp.zeros((L,), jnp.float32)
    nvec = d_out // L

    def zfill(i, carry):
        for r in range(nvec):
            rows_v[0, i, pl.ds(r * L, L)] = zero
        return carry

    lax.fori_loop(0, CHUNK, zfill, 0)

    def zcopy(k, carry):
        pltpu.sync_copy(rows_v.at[0], acc_sh.at[pl.ds(s * spt + k * CHUNK, CHUNK)])
        return carry

    lax.fori_loop(0, spt // CHUNK, zcopy, 0)
    pltpu.sync_copy(rows_hbm.at[pl.ds(wid * cbt, BLK)], row_b.at[0])
    pltpu.sync_copy(cols_hbm.at[pl.ds(wid * cbt, BLK)], col_b.at[0])
    pltpu.sync_copy(ew_hbm.at[pl.ds(wid * cbt, BLK)], ew_b.at[0])
    plsc.subcore_barrier()

    def drain(sem):
        pltpu.make_async_copy(g_hbm.at[row_b.at[0, 0]], rows_v.at[0], sem).wait()

    def gather(ib, j, buf):
        pltpu.async_copy(g_hbm.at[row_b.at[ib, j]], rows_v.at[buf], gsem)

    def block(i, carry):
        ib = i % 2
        @pl.when(i + 1 < nblk)
        def _():
            base = wid * cbt + (i + 1) * BLK
            pltpu.async_copy(rows_hbm.at[pl.ds(base, BLK)],
                             row_b.at[(i + 1) % 2], isem)
            pltpu.async_copy(cols_hbm.at[pl.ds(base, BLK)],
                             col_b.at[(i + 1) % 2], isem)
            pltpu.async_copy(ew_hbm.at[pl.ds(base, BLK)],
                             ew_b.at[(i + 1) % 2], isem)

        gather(ib, 0, 0)
        for j in range(BLK):
            drain(gsem)
            if j >= 1:
                drain(ssem)
            if j + 1 < BLK:
                gather(ib, j + 1, (j + 1) % 2)

            def grp(gi, carry2):
                wv = ew_b[ib, j, pl.ds(gi * L, L)]
                for u in range(L):
                    w_e = wv[u]
                    eidx = gi * L + u
                    for r in range(nvec):
                        sl = pl.ds(r * L, L)
                        rows_v[j % 2, eidx, sl] = rows_v[j % 2, eidx, sl] * w_e
                return carry2

            lax.fori_loop(0, CHUNK // L, grp, 0)
            pltpu.async_copy(rows_v.at[j % 2], acc_sh.at[col_b.at[ib, j]],
                             ssem, add=True)
        drain(ssem)

        @pl.when(i + 1 < nblk)
        def _():
            for _k in range(3):
                pltpu.make_async_copy(rows_hbm.at[pl.ds(0, BLK)],
                                      row_b.at[0], isem).wait()

        return carry

    lax.fori_loop(0, nblk, block, 0)
    plsc.subcore_barrier()
    n_pad = acc_sh.shape[0]
    pltpu.sync_copy(acc_sh.at[pl.ds(s * spt, spt)],
                    acc_out.at[pl.ds(c * n_pad + s * spt, spt)])


def _tc1_body(x_ref, w_ref, deg_ref, g_ref, dinv_ref):
    d = deg_ref[0] + deg_ref[1] + 1.0
    di = lax.rsqrt(d)
    h = jnp.dot(x_ref[...], w_ref[...], preferred_element_type=jnp.float32)
    g_ref[...] = h * di
    dinv_ref[...] = di


def _tc2_body(a_ref, g_ref, dinv_ref, b_ref, o_ref):
    o_ref[...] = (a_ref[0] + a_ref[1] + g_ref[...]) * dinv_ref[...] + b_ref[...]


def kernel(x, edge_index, edge_weight, W, b):
    f32 = jnp.float32
    n, d_in = x.shape
    d_out = W.shape[1]
    e = edge_index.shape[1]

    nw = NC * NS
    n_pad = -(-n // 512) * 512
    spt = n_pad // NS
    cpt = -(-e // (nw * KB)) * KB
    cbt = cpt // CHUNK
    e_pad = nw * cpt
    pad = e_pad - e

    row = edge_index[0]
    col = edge_index[1]
    pad_idx = jnp.arange(pad, dtype=jnp.int32) % n
    rows_p = jnp.concatenate([row, pad_idx]).reshape(-1, CHUNK)
    cols_p = jnp.concatenate([col, pad_idx]).reshape(-1, CHUNK)
    ew_p = jnp.concatenate([edge_weight, jnp.zeros((pad,), f32)]).reshape(-1, CHUNK)
    x_pad = jnp.pad(x, ((0, n_pad - n), (0, 0)))

    mesh = plsc.VectorSubcoreMesh(core_axis_name="c", subcore_axis_name="s")

    deg_fn = pl.kernel(
        functools.partial(_deg_body, cbt, spt),
        out_type=jax.ShapeDtypeStruct((NC * n_pad,), f32),
        mesh=mesh,
        scratch_types=[
            pltpu.VMEM((cbt, CHUNK), jnp.int32),
            pltpu.VMEM((cbt, CHUNK), f32),
            pltpu.VMEM((spt,), f32),
            pltpu.VMEM_SHARED((n_pad,), f32),
            pltpu.SemaphoreType.DMA,
        ],
    )
    deg_flat = deg_fn(cols_p, ew_p)
    deg3 = deg_flat.reshape(NC, n_pad, 1)

    rb = 1024
    grid = (n_pad // rb,)
    g, dinv = pl.pallas_call(
        _tc1_body,
        grid=grid,
        in_specs=[
            pl.BlockSpec((rb, d_in), lambda i: (i, 0)),
            pl.BlockSpec((d_in, d_out), lambda i: (0, 0)),
            pl.BlockSpec((NC, rb, 1), lambda i: (0, i, 0)),
        ],
        out_specs=[
            pl.BlockSpec((rb, d_out), lambda i: (i, 0)),
            pl.BlockSpec((rb, 1), lambda i: (i, 0)),
        ],
        out_shape=[
            jax.ShapeDtypeStruct((n_pad, d_out), f32),
            jax.ShapeDtypeStruct((n_pad, 1), f32),
        ],
    )(x_pad, W, deg3)

    acc_fn = pl.kernel(
        functools.partial(_acc_body, cbt, spt, d_out),
        out_type=jax.ShapeDtypeStruct((NC * n_pad, d_out), f32),
        mesh=mesh,
        scratch_types=[
            pltpu.VMEM((2, BLK, CHUNK), jnp.int32),
            pltpu.VMEM((2, BLK, CHUNK), jnp.int32),
            pltpu.VMEM((2, BLK, CHUNK), f32),
            pltpu.VMEM((2, CHUNK, d_out), f32),
            pltpu.VMEM_SHARED((n_pad, d_out), f32),
            pltpu.SemaphoreType.DMA,
            pltpu.SemaphoreType.DMA,
            pltpu.SemaphoreType.DMA,
        ],
    )
    acc = acc_fn(g, rows_p, cols_p, ew_p).reshape(NC, n_pad, d_out)

    out_pad = pl.pallas_call(
        _tc2_body,
        grid=grid,
        in_specs=[
            pl.BlockSpec((NC, rb, d_out), lambda i: (0, i, 0)),
            pl.BlockSpec((rb, d_out), lambda i: (i, 0)),
            pl.BlockSpec((rb, 1), lambda i: (i, 0)),
            pl.BlockSpec((1, d_out), lambda i: (0, 0)),
        ],
        out_specs=pl.BlockSpec((rb, d_out), lambda i: (i, 0)),
        out_shape=jax.ShapeDtypeStruct((n_pad, d_out), f32),
    )(acc, g, dinv, b.reshape(1, d_out))

    return out_pad[:n]


# speedup vs baseline: 33.6119x; 1.0042x over previous
import functools

import jax
import jax.numpy as jnp
from jax import lax
from jax.experimental import pallas as pl
from jax.experimental.pallas import tpu as pltpu
from jax.experimental.pallas import tpu_sc as plsc

NC = 2
NS = 16
L = 16
CHUNK = 128
BLK = 8
KB = CHUNK * BLK


def _deg_body(cbt, spt, cols_hbm, ew_hbm, deg_out, col_a, ew_a,
              stripe_v, deg_sh, ssem):
    c = lax.axis_index("c")
    s = lax.axis_index("s")
    wid = s * NC + c
    zero = jnp.zeros((L,), jnp.float32)

    def zfill(i, carry):
        stripe_v[pl.ds(i * L, L)] = zero
        return carry

    lax.fori_loop(0, spt // L, zfill, 0)
    pltpu.sync_copy(stripe_v, deg_sh.at[pl.ds(s * spt, spt)])
    pltpu.sync_copy(cols_hbm.at[pl.ds(wid * cbt, cbt)], col_a)
    pltpu.sync_copy(ew_hbm.at[pl.ds(wid * cbt, cbt)], ew_a)
    plsc.subcore_barrier()

    lag = 8

    def drain():
        pltpu.make_async_copy(ew_hbm.at[0], ew_a.at[0], ssem).wait()

    def chunk(j, carry):
        pltpu.async_copy(ew_a.at[j], deg_sh.at[col_a.at[j]], ssem, add=True)
        pl.when(j >= lag)(drain)
        return carry

    lax.fori_loop(0, cbt, chunk, 0)

    def drain_i(i, carry):
        drain()
        return carry

    lax.fori_loop(0, lag, drain_i, 0)
    plsc.subcore_barrier()
    n_pad = deg_sh.shape[0]
    pltpu.sync_copy(deg_sh.at[pl.ds(s * spt, spt)],
                    deg_out.at[pl.ds(c * n_pad + s * spt, spt)])


def _acc_body(cbt, spt, d_out, g_hbm, rows_hbm, cols_hbm, ew_hbm,
              acc_out, row_b, col_b, ew_b, rows_v, acc_sh, gsem, ssem, isem):
    c = lax.axis_index("c")
    s = lax.axis_index("s")
    wid = s * NC + c
    nblk = cbt // BLK
    zero = jnp.zeros((L,), jnp.float32)
    nvec = d_out // L

    def zfill(i, carry):
        for r in range(nvec):
            rows_v[0, i, pl.ds(r * L, L)] = zero
        return carry

    lax.fori_loop(0, CHUNK, zfill, 0)

    def zcopy(k, carry):
        pltpu.sync_copy(rows_v.at[0], acc_sh.at[pl.ds(s * spt + k * CHUNK, CHUNK)])
        return carry

    lax.fori_loop(0, spt // CHUNK, zcopy, 0)
    pltpu.sync_copy(rows_hbm.at[pl.ds(wid * cbt, BLK)], row_b.at[0])
    pltpu.sync_copy(cols_hbm.at[pl.ds(wid * cbt, BLK)], col_b.at[0])
    pltpu.sync_copy(ew_hbm.at[pl.ds(wid * cbt, BLK)], ew_b.at[0])
    plsc.subcore_barrier()

    def drain(sem):
        pltpu.make_async_copy(g_hbm.at[row_b.at[0, 0]], rows_v.at[0], sem).wait()

    def gather(ib, j, buf):
        pltpu.async_copy(g_hbm.at[row_b.at[ib, j]], rows_v.at[buf], gsem)

    def block(i, carry):
        ib = i % 2
        @pl.when(i + 1 < nblk)
        def _():
            base = wid * cbt + (i + 1) * BLK
            pltpu.async_copy(rows_hbm.at[pl.ds(base, BLK)],
                             row_b.at[(i + 1) % 2], isem)
            pltpu.async_copy(cols_hbm.at[pl.ds(base, BLK)],
                             col_b.at[(i + 1) % 2], isem)
            pltpu.async_copy(ew_hbm.at[pl.ds(base, BLK)],
                             ew_b.at[(i + 1) % 2], isem)

        gather(ib, 0, 0)
        for j in range(BLK):
            if j >= 1:
                drain(ssem)
            if j + 1 < BLK:
                gather(ib, j + 1, (j + 1) % 2)
            drain(gsem)

            def grp(gi, carry2):
                wv = ew_b[ib, j, pl.ds(gi * L, L)]
                for u in range(L):
                    w_e = wv[u]
                    eidx = gi * L + u
                    for r in range(nvec):
                        sl = pl.ds(r * L, L)
                        rows_v[j % 2, eidx, sl] = rows_v[j % 2, eidx, sl] * w_e
                return carry2

            lax.fori_loop(0, CHUNK // L, grp, 0)
            pltpu.async_copy(rows_v.at[j % 2], acc_sh.at[col_b.at[ib, j]],
                             ssem, add=True)
        drain(ssem)

        @pl.when(i + 1 < nblk)
        def _():
            for _k in range(3):
                pltpu.make_async_copy(rows_hbm.at[pl.ds(0, BLK)],
                                      row_b.at[0], isem).wait()

        return carry

    lax.fori_loop(0, nblk, block, 0)
    plsc.subcore_barrier()
    n_pad = acc_sh.shape[0]
    pltpu.sync_copy(acc_sh.at[pl.ds(s * spt, spt)],
                    acc_out.at[pl.ds(c * n_pad + s * spt, spt)])


def _tc1_body(x_ref, w_ref, deg_ref, g_ref, dinv_ref):
    d = deg_ref[0] + deg_ref[1] + 1.0
    di = lax.rsqrt(d)
    h = jnp.dot(x_ref[...], w_ref[...], preferred_element_type=jnp.float32)
    g_ref[...] = h * di
    dinv_ref[...] = di


def _tc2_body(a_ref, g_ref, dinv_ref, b_ref, o_ref):
    o_ref[...] = (a_ref[0] + a_ref[1] + g_ref[...]) * dinv_ref[...] + b_ref[...]


def kernel(x, edge_index, edge_weight, W, b):
    f32 = jnp.float32
    n, d_in = x.shape
    d_out = W.shape[1]
    e = edge_index.shape[1]

    nw = NC * NS
    n_pad = -(-n // 512) * 512
    spt = n_pad // NS
    cpt = -(-e // (nw * KB)) * KB
    cbt = cpt // CHUNK
    e_pad = nw * cpt
    pad = e_pad - e

    row = edge_index[0]
    col = edge_index[1]
    pad_idx = jnp.arange(pad, dtype=jnp.int32) % n
    rows_p = jnp.concatenate([row, pad_idx]).reshape(-1, CHUNK)
    cols_p = jnp.concatenate([col, pad_idx]).reshape(-1, CHUNK)
    ew_p = jnp.concatenate([edge_weight, jnp.zeros((pad,), f32)]).reshape(-1, CHUNK)
    x_pad = jnp.pad(x, ((0, n_pad - n), (0, 0)))

    mesh = plsc.VectorSubcoreMesh(core_axis_name="c", subcore_axis_name="s")

    deg_fn = pl.kernel(
        functools.partial(_deg_body, cbt, spt),
        out_type=jax.ShapeDtypeStruct((NC * n_pad,), f32),
        mesh=mesh,
        scratch_types=[
            pltpu.VMEM((cbt, CHUNK), jnp.int32),
            pltpu.VMEM((cbt, CHUNK), f32),
            pltpu.VMEM((spt,), f32),
            pltpu.VMEM_SHARED((n_pad,), f32),
            pltpu.SemaphoreType.DMA,
        ],
    )
    deg_flat = deg_fn(cols_p, ew_p)
    deg3 = deg_flat.reshape(NC, n_pad, 1)

    rb = 1024
    grid = (n_pad // rb,)
    g, dinv = pl.pallas_call(
        _tc1_body,
        grid=grid,
        in_specs=[
            pl.BlockSpec((rb, d_in), lambda i: (i, 0)),
            pl.BlockSpec((d_in, d_out), lambda i: (0, 0)),
            pl.BlockSpec((NC, rb, 1), lambda i: (0, i, 0)),
        ],
        out_specs=[
            pl.BlockSpec((rb, d_out), lambda i: (i, 0)),
            pl.BlockSpec((rb, 1), lambda i: (i, 0)),
        ],
        out_shape=[
            jax.ShapeDtypeStruct((n_pad, d_out), f32),
            jax.ShapeDtypeStruct((n_pad, 1), f32),
        ],
    )(x_pad, W, deg3)

    acc_fn = pl.kernel(
        functools.partial(_acc_body, cbt, spt, d_out),
        out_type=jax.ShapeDtypeStruct((NC * n_pad, d_out), f32),
        mesh=mesh,
        scratch_types=[
            pltpu.VMEM((2, BLK, CHUNK), jnp.int32),
            pltpu.VMEM((2, BLK, CHUNK), jnp.int32),
            pltpu.VMEM((2, BLK, CHUNK), f32),
            pltpu.VMEM((2, CHUNK, d_out), f32),
            pltpu.VMEM_SHARED((n_pad, d_out), f32),
            pltpu.SemaphoreType.DMA,
            pltpu.SemaphoreType.DMA,
            pltpu.SemaphoreType.DMA,
        ],
    )
    acc = acc_fn(g, rows_p, cols_p, ew_p).reshape(NC, n_pad, d_out)

    out_pad = pl.pallas_call(
        _tc2_body,
        grid=grid,
        in_specs=[
            pl.BlockSpec((NC, rb, d_out), lambda i: (0, i, 0)),
            pl.BlockSpec((rb, d_out), lambda i: (i, 0)),
            pl.BlockSpec((rb, 1), lambda i: (i, 0)),
            pl.BlockSpec((1, d_out), lambda i: (0, 0)),
        ],
        out_specs=pl.BlockSpec((rb, d_out), lambda i: (i, 0)),
        out_shape=jax.ShapeDtypeStruct((n_pad, d_out), f32),
    )(acc, g, dinv, b.reshape(1, d_out))

    return out_pad[:n]
